# baseline (speedup 1.0000x reference)
import functools

import jax
import jax.numpy as jnp
from jax import lax
from jax.experimental import pallas as pl
from jax.experimental.pallas import tpu as pltpu
from jax.experimental.pallas import tpu_sc as plsc

N = 10000
E = 160000
NPAD = 10240
NC, NS = 2, 16
NW = NC * NS
EPT = E // NW
CHUNK = 125
NCHUNK = EPT // CHUNK
RPT = NPAD // NS

_F32 = jnp.float32


NBUF = 4
NGRP = NCHUNK // NBUF


_SPT = N // NS


def _make_sc_aggregate(D):
  mesh = plsc.VectorSubcoreMesh(core_axis_name="c", subcore_axis_name="s")

  @functools.partial(
      pl.kernel,
      out_type=jax.ShapeDtypeStruct((NC, NPAD, D), _F32),
      mesh=mesh,
      compiler_params=pltpu.CompilerParams(use_tc_tiling_on_sc=False),
      scratch_types=[
          pltpu.VMEM((NCHUNK, CHUNK), jnp.int32),
          pltpu.VMEM((NCHUNK, CHUNK), jnp.int32),
          [pltpu.VMEM((CHUNK, D), _F32) for _ in range(NBUF)],
          [pltpu.SemaphoreType.DMA for _ in range(NBUF)],
          [pltpu.SemaphoreType.DMA for _ in range(NBUF)],
          pltpu.VMEM_SHARED((N, D), _F32),
          pltpu.VMEM_SHARED((NPAD, D), _F32),
      ],
  )
  def agg(table_hbm, srcs_hbm, dsts_hbm, zeros_hbm, out_hbm,
          src_v, dst_v, bufs, gsems, ssems, tab_sh, acc_sh):
    c = lax.axis_index("c")
    s = lax.axis_index("s")
    wid = s * NC + c
    pltpu.sync_copy(table_hbm.at[pl.ds(s * _SPT, _SPT)],
                    tab_sh.at[pl.ds(s * _SPT, _SPT)])
    pltpu.sync_copy(zeros_hbm.at[pl.ds(s * RPT, RPT)],
                    acc_sh.at[pl.ds(s * RPT, RPT)])
    pltpu.sync_copy(srcs_hbm.at[pl.ds(wid * NCHUNK, NCHUNK)], src_v)
    pltpu.sync_copy(dsts_hbm.at[pl.ds(wid * NCHUNK, NCHUNK)], dst_v)
    plsc.subcore_barrier()

    for b in range(NBUF):
      pltpu.async_copy(tab_sh.at[src_v.at[b]], bufs[b], gsems[b])

    def body(g, carry):
      for b in range(NBUF):
        j = g * NBUF + b
        pltpu.make_async_copy(tab_sh.at[src_v.at[j]], bufs[b],
                              gsems[b]).wait()
        pltpu.async_copy(bufs[b], acc_sh.at[dst_v.at[j]], ssems[b], add=True)
      for b in range(NBUF):
        j = g * NBUF + b
        pltpu.make_async_copy(bufs[b], acc_sh.at[dst_v.at[j]],
                              ssems[b]).wait()

        @pl.when(g + 1 < NGRP)
        def _():
          pltpu.async_copy(tab_sh.at[src_v.at[j + NBUF]], bufs[b],
                           gsems[b])
      return carry

    lax.fori_loop(0, NGRP, body, 0)
    plsc.subcore_barrier()
    pltpu.sync_copy(acc_sh.at[pl.ds(s * RPT, RPT)],
                    out_hbm.at[c].at[pl.ds(s * RPT, RPT)])

  return agg


_sc_agg64 = _make_sc_aggregate(64)
_sc_agg16 = _make_sc_aggregate(16)

_deg_mesh = plsc.VectorSubcoreMesh(core_axis_name="c", subcore_axis_name="s")


@functools.partial(
    pl.kernel,
    out_type=jax.ShapeDtypeStruct((NC, NPAD, 8), _F32),
    mesh=_deg_mesh,
    compiler_params=pltpu.CompilerParams(use_tc_tiling_on_sc=False),
    scratch_types=[
        pltpu.VMEM((NCHUNK, CHUNK), jnp.int32),
        pltpu.VMEM((CHUNK, 8), _F32),
        pltpu.VMEM_SHARED((NPAD, 8), _F32),
        pltpu.SemaphoreType.DMA,
    ],
)
def _sc_degree(dsts_hbm, ones_hbm, zeros_hbm, out_hbm,
               dst_v, ones_v, deg_sh, sem):
  c = lax.axis_index("c")
  s = lax.axis_index("s")
  wid = s * NC + c
  pltpu.sync_copy(zeros_hbm.at[pl.ds(s * RPT, RPT)],
                  deg_sh.at[pl.ds(s * RPT, RPT)])
  pltpu.sync_copy(ones_hbm, ones_v)
  pltpu.sync_copy(dsts_hbm.at[pl.ds(wid * NCHUNK, NCHUNK)], dst_v)
  plsc.subcore_barrier()

  def body(j, carry):
    pltpu.async_copy(ones_v, deg_sh.at[dst_v.at[j]], sem, add=True)
    return carry

  lax.fori_loop(0, NCHUNK, body, 0)

  def drain(j, carry):
    pltpu.make_async_copy(ones_v, deg_sh.at[dst_v.at[j]], sem).wait()
    return carry

  lax.fori_loop(0, NCHUNK, drain, 0)
  plsc.subcore_barrier()
  pltpu.sync_copy(deg_sh.at[pl.ds(s * RPT, RPT)],
                  out_hbm.at[c].at[pl.ds(s * RPT, RPT)])


_BLK = 1000
_NBLK = N // _BLK


def _tcmm_body(x_ref, w_ref, xw_ref):
  xw_ref[...] = jnp.dot(x_ref[...], w_ref[...], preferred_element_type=_F32)


_tc_mm = pl.pallas_call(
    _tcmm_body,
    grid=(_NBLK,),
    in_specs=[
        pl.BlockSpec((_BLK, 256), lambda i: (i, 0)),
        pl.BlockSpec((256, 64), lambda i: (0, 0)),
    ],
    out_specs=pl.BlockSpec((_BLK, 64), lambda i: (i, 0)),
    out_shape=jax.ShapeDtypeStruct((N, 64), _F32),
)


def _tcs_body(xw_ref, d_ref, y_ref, dis_ref):
  deg = d_ref[0, :, 0:1] + d_ref[1, :, 0:1] + 1.0
  dis = lax.rsqrt(deg)
  y_ref[...] = xw_ref[...] * dis
  dis_ref[...] = dis


_tc_scale = pl.pallas_call(
    _tcs_body,
    grid=(_NBLK,),
    in_specs=[
        pl.BlockSpec((_BLK, 64), lambda i: (i, 0)),
        pl.BlockSpec((2, _BLK, 8), lambda i: (0, i, 0)),
    ],
    out_specs=[
        pl.BlockSpec((_BLK, 64), lambda i: (i, 0)),
        pl.BlockSpec((_BLK, 1), lambda i: (i, 0)),
    ],
    out_shape=[
        jax.ShapeDtypeStruct((N, 64), _F32),
        jax.ShapeDtypeStruct((N, 1), _F32),
    ],
)


def _tcb_body(p_ref, y1_ref, dis_ref, b1_ref, w2_ref, y2_ref):
  dis = dis_ref[...]
  acc = p_ref[0] + p_ref[1] + y1_ref[...]
  z = acc * dis + b1_ref[...]
  h = jnp.where(z >= 0, z, 0.01 * z)
  y2_ref[...] = jnp.dot(h, w2_ref[...], preferred_element_type=_F32) * dis


_tc_b = pl.pallas_call(
    _tcb_body,
    grid=(_NBLK,),
    in_specs=[
        pl.BlockSpec((2, _BLK, 64), lambda i: (0, i, 0)),
        pl.BlockSpec((_BLK, 64), lambda i: (i, 0)),
        pl.BlockSpec((_BLK, 1), lambda i: (i, 0)),
        pl.BlockSpec((1, 64), lambda i: (0, 0)),
        pl.BlockSpec((64, 16), lambda i: (0, 0)),
    ],
    out_specs=pl.BlockSpec((_BLK, 16), lambda i: (i, 0)),
    out_shape=jax.ShapeDtypeStruct((N, 16), _F32),
)


def _tcc_body(q_ref, y2_ref, dis_ref, b2_ref, o_ref):
  z = (q_ref[0] + q_ref[1] + y2_ref[...]) * dis_ref[...] + b2_ref[...]
  m = jnp.max(z, axis=1, keepdims=True)
  e = jnp.exp(z - m)
  o_ref[...] = (e / jnp.sum(e, axis=1, keepdims=True))[:, 0:4]


_tc_c = pl.pallas_call(
    _tcc_body,
    grid=(_NBLK,),
    in_specs=[
        pl.BlockSpec((2, _BLK, 16), lambda i: (0, i, 0)),
        pl.BlockSpec((_BLK, 16), lambda i: (i, 0)),
        pl.BlockSpec((_BLK, 1), lambda i: (i, 0)),
        pl.BlockSpec((1, 16), lambda i: (0, 0)),
    ],
    out_specs=pl.BlockSpec((_BLK, 4), lambda i: (i, 0)),
    out_shape=jax.ShapeDtypeStruct((N, 4), _F32),
)


def kernel(x, edge_index, W1, b1, W2, b2):
  srcs = edge_index[0].reshape(NW * NCHUNK, CHUNK)
  dsts = edge_index[1].reshape(NW * NCHUNK, CHUNK)
  zeros16 = jnp.zeros((NPAD, 16), _F32)
  zeros64 = jnp.zeros((NPAD, 64), _F32)
  zeros8 = jnp.zeros((NPAD, 8), _F32)
  ones8 = jnp.ones((CHUNK, 8), _F32)
  w2pad = jnp.pad(W2, ((0, 0), (0, 12)))
  b2pad = jnp.concatenate([b2, jnp.full((12,), -1e30, _F32)]).reshape(1, 16)

  d = _sc_degree(dsts, ones8, zeros8)
  xw = _tc_mm(x, W1)
  y1, dis = _tc_scale(xw, d)
  p = _sc_agg64(y1, srcs, dsts, zeros64)
  y2 = _tc_b(p, y1, dis, b1.reshape(1, 64), w2pad)
  q = _sc_agg16(y2, srcs, dsts, zeros16)
  return _tc_c(q, y2, dis, b2pad)


# speedup vs baseline: 1.0784x; 1.0784x over previous
import functools

import jax
import jax.numpy as jnp
from jax import lax
from jax.experimental import pallas as pl
from jax.experimental.pallas import tpu as pltpu
from jax.experimental.pallas import tpu_sc as plsc

N = 10000
E = 160000
NPAD = 10240
NC, NS = 2, 16
NW = NC * NS
EPT = E // NW
CHUNK = 125
NCHUNK = EPT // CHUNK
RPT = NPAD // NS

_F32 = jnp.float32


NBUF = 4
NGRP = NCHUNK // NBUF


_SPT = N // NS


def _make_sc_aggregate(D, stage, nbuf):
  mesh = plsc.VectorSubcoreMesh(core_axis_name="c", subcore_axis_name="s")
  ngrp = NCHUNK // nbuf

  @functools.partial(
      pl.kernel,
      out_type=jax.ShapeDtypeStruct((NC, NPAD, D), _F32),
      mesh=mesh,
      compiler_params=pltpu.CompilerParams(use_tc_tiling_on_sc=False),
      scratch_types=[
          pltpu.VMEM((NCHUNK, CHUNK), jnp.int32),
          pltpu.VMEM((NCHUNK, CHUNK), jnp.int32),
          [pltpu.VMEM((CHUNK, D), _F32) for _ in range(nbuf)],
          [pltpu.SemaphoreType.DMA for _ in range(nbuf)],
          [pltpu.SemaphoreType.DMA for _ in range(nbuf)],
      ] + ([pltpu.VMEM_SHARED((N, D), _F32)] if stage else []) + [
          pltpu.VMEM_SHARED((NPAD, D), _F32),
      ],
  )
  def agg(table_hbm, srcs_hbm, dsts_hbm, zeros_hbm, out_hbm,
          src_v, dst_v, bufs, gsems, ssems, *rest):
    tab_sh = rest[0] if stage else None
    acc_sh = rest[-1]
    c = lax.axis_index("c")
    s = lax.axis_index("s")
    wid = s * NC + c
    table = tab_sh if stage else table_hbm
    if stage:
      pltpu.sync_copy(table_hbm.at[pl.ds(s * _SPT, _SPT)],
                      tab_sh.at[pl.ds(s * _SPT, _SPT)])
    pltpu.sync_copy(zeros_hbm.at[pl.ds(s * RPT, RPT)],
                    acc_sh.at[pl.ds(s * RPT, RPT)])
    pltpu.sync_copy(srcs_hbm.at[pl.ds(wid * NCHUNK, NCHUNK)], src_v)
    pltpu.sync_copy(dsts_hbm.at[pl.ds(wid * NCHUNK, NCHUNK)], dst_v)
    plsc.subcore_barrier()

    for b in range(nbuf):
      pltpu.async_copy(table.at[src_v.at[b]], bufs[b], gsems[b])

    def body(g, carry):
      for b in range(nbuf):
        j = g * nbuf + b
        pltpu.make_async_copy(table.at[src_v.at[j]], bufs[b],
                              gsems[b]).wait()
        pltpu.async_copy(bufs[b], acc_sh.at[dst_v.at[j]], ssems[b], add=True)
      for b in range(nbuf):
        j = g * nbuf + b
        pltpu.make_async_copy(bufs[b], acc_sh.at[dst_v.at[j]],
                              ssems[b]).wait()

        @pl.when(g + 1 < ngrp)
        def _():
          pltpu.async_copy(table.at[src_v.at[j + nbuf]], bufs[b],
                           gsems[b])
      return carry

    lax.fori_loop(0, ngrp, body, 0)
    plsc.subcore_barrier()
    pltpu.sync_copy(acc_sh.at[pl.ds(s * RPT, RPT)],
                    out_hbm.at[c].at[pl.ds(s * RPT, RPT)])

  return agg


_sc_agg64 = _make_sc_aggregate(64, stage=False, nbuf=8)
_sc_agg16 = _make_sc_aggregate(16, stage=True, nbuf=4)

_deg_mesh = plsc.VectorSubcoreMesh(core_axis_name="c", subcore_axis_name="s")


@functools.partial(
    pl.kernel,
    out_type=jax.ShapeDtypeStruct((NC, NPAD, 8), _F32),
    mesh=_deg_mesh,
    compiler_params=pltpu.CompilerParams(use_tc_tiling_on_sc=False),
    scratch_types=[
        pltpu.VMEM((NCHUNK, CHUNK), jnp.int32),
        pltpu.VMEM((CHUNK, 8), _F32),
        pltpu.VMEM_SHARED((NPAD, 8), _F32),
        pltpu.SemaphoreType.DMA,
    ],
)
def _sc_degree(dsts_hbm, ones_hbm, zeros_hbm, out_hbm,
               dst_v, ones_v, deg_sh, sem):
  c = lax.axis_index("c")
  s = lax.axis_index("s")
  wid = s * NC + c
  pltpu.sync_copy(zeros_hbm.at[pl.ds(s * RPT, RPT)],
                  deg_sh.at[pl.ds(s * RPT, RPT)])
  pltpu.sync_copy(ones_hbm, ones_v)
  pltpu.sync_copy(dsts_hbm.at[pl.ds(wid * NCHUNK, NCHUNK)], dst_v)
  plsc.subcore_barrier()

  def body(j, carry):
    pltpu.async_copy(ones_v, deg_sh.at[dst_v.at[j]], sem, add=True)
    return carry

  lax.fori_loop(0, NCHUNK, body, 0)

  def drain(j, carry):
    pltpu.make_async_copy(ones_v, deg_sh.at[dst_v.at[j]], sem).wait()
    return carry

  lax.fori_loop(0, NCHUNK, drain, 0)
  plsc.subcore_barrier()
  pltpu.sync_copy(deg_sh.at[pl.ds(s * RPT, RPT)],
                  out_hbm.at[c].at[pl.ds(s * RPT, RPT)])


_BLK = 1000
_NBLK = N // _BLK


def _tcmm_body(x_ref, w_ref, xw_ref):
  xw_ref[...] = jnp.dot(x_ref[...], w_ref[...], preferred_element_type=_F32)


_tc_mm = pl.pallas_call(
    _tcmm_body,
    grid=(_NBLK,),
    in_specs=[
        pl.BlockSpec((_BLK, 256), lambda i: (i, 0)),
        pl.BlockSpec((256, 64), lambda i: (0, 0)),
    ],
    out_specs=pl.BlockSpec((_BLK, 64), lambda i: (i, 0)),
    out_shape=jax.ShapeDtypeStruct((N, 64), _F32),
)


def _tcs_body(xw_ref, d_ref, y_ref, dis_ref):
  deg = d_ref[0, :, 0:1] + d_ref[1, :, 0:1] + 1.0
  dis = lax.rsqrt(deg)
  y_ref[...] = xw_ref[...] * dis
  dis_ref[...] = dis


_tc_scale = pl.pallas_call(
    _tcs_body,
    grid=(_NBLK,),
    in_specs=[
        pl.BlockSpec((_BLK, 64), lambda i: (i, 0)),
        pl.BlockSpec((2, _BLK, 8), lambda i: (0, i, 0)),
    ],
    out_specs=[
        pl.BlockSpec((_BLK, 64), lambda i: (i, 0)),
        pl.BlockSpec((_BLK, 1), lambda i: (i, 0)),
    ],
    out_shape=[
        jax.ShapeDtypeStruct((N, 64), _F32),
        jax.ShapeDtypeStruct((N, 1), _F32),
    ],
)


def _tcb_body(p_ref, y1_ref, dis_ref, b1_ref, w2_ref, y2_ref):
  dis = dis_ref[...]
  acc = p_ref[0] + p_ref[1] + y1_ref[...]
  z = acc * dis + b1_ref[...]
  h = jnp.where(z >= 0, z, 0.01 * z)
  y2_ref[...] = jnp.dot(h, w2_ref[...], preferred_element_type=_F32) * dis


_tc_b = pl.pallas_call(
    _tcb_body,
    grid=(_NBLK,),
    in_specs=[
        pl.BlockSpec((2, _BLK, 64), lambda i: (0, i, 0)),
        pl.BlockSpec((_BLK, 64), lambda i: (i, 0)),
        pl.BlockSpec((_BLK, 1), lambda i: (i, 0)),
        pl.BlockSpec((1, 64), lambda i: (0, 0)),
        pl.BlockSpec((64, 16), lambda i: (0, 0)),
    ],
    out_specs=pl.BlockSpec((_BLK, 16), lambda i: (i, 0)),
    out_shape=jax.ShapeDtypeStruct((N, 16), _F32),
)


def _tcc_body(q_ref, y2_ref, dis_ref, b2_ref, o_ref):
  z = (q_ref[0] + q_ref[1] + y2_ref[...]) * dis_ref[...] + b2_ref[...]
  m = jnp.max(z, axis=1, keepdims=True)
  e = jnp.exp(z - m)
  o_ref[...] = (e / jnp.sum(e, axis=1, keepdims=True))[:, 0:4]


_tc_c = pl.pallas_call(
    _tcc_body,
    grid=(_NBLK,),
    in_specs=[
        pl.BlockSpec((2, _BLK, 16), lambda i: (0, i, 0)),
        pl.BlockSpec((_BLK, 16), lambda i: (i, 0)),
        pl.BlockSpec((_BLK, 1), lambda i: (i, 0)),
        pl.BlockSpec((1, 16), lambda i: (0, 0)),
    ],
    out_specs=pl.BlockSpec((_BLK, 4), lambda i: (i, 0)),
    out_shape=jax.ShapeDtypeStruct((N, 4), _F32),
)


def kernel(x, edge_index, W1, b1, W2, b2):
  srcs = edge_index[0].reshape(NW * NCHUNK, CHUNK)
  dsts = edge_index[1].reshape(NW * NCHUNK, CHUNK)
  zeros16 = jnp.zeros((NPAD, 16), _F32)
  zeros64 = jnp.zeros((NPAD, 64), _F32)
  zeros8 = jnp.zeros((NPAD, 8), _F32)
  ones8 = jnp.ones((CHUNK, 8), _F32)
  w2pad = jnp.pad(W2, ((0, 0), (0, 12)))
  b2pad = jnp.concatenate([b2, jnp.full((12,), -1e30, _F32)]).reshape(1, 16)

  d = _sc_degree(dsts, ones8, zeros8)
  xw = _tc_mm(x, W1)
  y1, dis = _tc_scale(xw, d)
  p = _sc_agg64(y1, srcs, dsts, zeros64)
  y2 = _tc_b(p, y1, dis, b1.reshape(1, 64), w2pad)
  q = _sc_agg16(y2, srcs, dsts, zeros16)
  return _tc_c(q, y2, dis, b2pad)


# speedup vs baseline: 1.1298x; 1.0477x over previous
import functools

import jax
import jax.numpy as jnp
from jax import lax
from jax.experimental import pallas as pl
from jax.experimental.pallas import tpu as pltpu
from jax.experimental.pallas import tpu_sc as plsc

N = 10000
E = 160000
NPAD = 10240
NC, NS = 2, 16
NW = NC * NS
EPT = E // NW
CHUNK = 125
NCHUNK = EPT // CHUNK
RPT = NPAD // NS

_F32 = jnp.float32


NBUF = 4
NGRP = NCHUNK // NBUF


_SPT = N // NS


def _make_sc_aggregate(D, stage, nbuf):
  mesh = plsc.VectorSubcoreMesh(core_axis_name="c", subcore_axis_name="s")
  ngrp = NCHUNK // nbuf

  @functools.partial(
      pl.kernel,
      out_type=jax.ShapeDtypeStruct((NC, NPAD, D), _F32),
      mesh=mesh,
      compiler_params=pltpu.CompilerParams(use_tc_tiling_on_sc=False),
      scratch_types=[
          pltpu.VMEM((NCHUNK, CHUNK), jnp.int32),
          pltpu.VMEM((NCHUNK, CHUNK), jnp.int32),
          [pltpu.VMEM((CHUNK, D), _F32) for _ in range(nbuf)],
          [pltpu.SemaphoreType.DMA for _ in range(nbuf)],
          [pltpu.SemaphoreType.DMA for _ in range(nbuf)],
      ] + ([pltpu.VMEM_SHARED((N, D), _F32)] if stage else []) + [
          pltpu.VMEM_SHARED((NPAD, D), _F32),
      ],
  )
  def agg(table_hbm, srcs_hbm, dsts_hbm, zeros_hbm, out_hbm,
          src_v, dst_v, bufs, gsems, ssems, *rest):
    tab_sh = rest[0] if stage else None
    acc_sh = rest[-1]
    c = lax.axis_index("c")
    s = lax.axis_index("s")
    wid = s * NC + c
    table = tab_sh if stage else table_hbm
    if stage:
      pltpu.sync_copy(table_hbm.at[pl.ds(s * _SPT, _SPT)],
                      tab_sh.at[pl.ds(s * _SPT, _SPT)])
    pltpu.sync_copy(zeros_hbm.at[pl.ds(s * RPT, RPT)],
                    acc_sh.at[pl.ds(s * RPT, RPT)])
    pltpu.sync_copy(srcs_hbm.at[pl.ds(wid * NCHUNK, NCHUNK)], src_v)
    pltpu.sync_copy(dsts_hbm.at[pl.ds(wid * NCHUNK, NCHUNK)], dst_v)
    plsc.subcore_barrier()

    for b in range(nbuf):
      pltpu.async_copy(table.at[src_v.at[b]], bufs[b], gsems[b])

    def body(g, carry):
      for b in range(nbuf):
        j = g * nbuf + b
        pltpu.make_async_copy(table.at[src_v.at[j]], bufs[b],
                              gsems[b]).wait()
        pltpu.async_copy(bufs[b], acc_sh.at[dst_v.at[j]], ssems[b], add=True)
      for b in range(nbuf):
        j = g * nbuf + b
        pltpu.make_async_copy(bufs[b], acc_sh.at[dst_v.at[j]],
                              ssems[b]).wait()

        @pl.when(g + 1 < ngrp)
        def _():
          pltpu.async_copy(table.at[src_v.at[j + nbuf]], bufs[b],
                           gsems[b])
      return carry

    lax.fori_loop(0, ngrp, body, 0)
    plsc.subcore_barrier()
    pltpu.sync_copy(acc_sh.at[pl.ds(s * RPT, RPT)],
                    out_hbm.at[c].at[pl.ds(s * RPT, RPT)])

  return agg


_sc_agg64 = _make_sc_aggregate(64, stage=False, nbuf=8)
_sc_agg16 = _make_sc_aggregate(16, stage=True, nbuf=4)

_deg_mesh = plsc.VectorSubcoreMesh(core_axis_name="c", subcore_axis_name="s")


@functools.partial(
    pl.kernel,
    out_type=jax.ShapeDtypeStruct((NC, NPAD, 8), _F32),
    mesh=_deg_mesh,
    compiler_params=pltpu.CompilerParams(use_tc_tiling_on_sc=False),
    scratch_types=[
        pltpu.VMEM((NCHUNK, CHUNK), jnp.int32),
        pltpu.VMEM((CHUNK, 8), _F32),
        pltpu.VMEM_SHARED((NPAD, 8), _F32),
        pltpu.SemaphoreType.DMA,
    ],
)
def _sc_degree(dsts_hbm, ones_hbm, zeros_hbm, out_hbm,
               dst_v, ones_v, deg_sh, sem):
  c = lax.axis_index("c")
  s = lax.axis_index("s")
  wid = s * NC + c
  pltpu.sync_copy(zeros_hbm.at[pl.ds(s * RPT, RPT)],
                  deg_sh.at[pl.ds(s * RPT, RPT)])
  pltpu.sync_copy(ones_hbm, ones_v)
  pltpu.sync_copy(dsts_hbm.at[pl.ds(wid * NCHUNK, NCHUNK)], dst_v)
  plsc.subcore_barrier()

  def body(j, carry):
    pltpu.async_copy(ones_v, deg_sh.at[dst_v.at[j]], sem, add=True)
    return carry

  lax.fori_loop(0, NCHUNK, body, 0)

  def drain(j, carry):
    pltpu.make_async_copy(ones_v, deg_sh.at[dst_v.at[j]], sem).wait()
    return carry

  lax.fori_loop(0, NCHUNK, drain, 0)
  plsc.subcore_barrier()
  pltpu.sync_copy(deg_sh.at[pl.ds(s * RPT, RPT)],
                  out_hbm.at[c].at[pl.ds(s * RPT, RPT)])


_BLK = 2000
_NBLK = N // _BLK


def _tcmm_body(x_ref, w_ref, xw_ref):
  xw_ref[...] = jnp.dot(x_ref[...], w_ref[...], preferred_element_type=_F32)


_tc_mm = pl.pallas_call(
    _tcmm_body,
    grid=(_NBLK,),
    in_specs=[
        pl.BlockSpec((_BLK, 256), lambda i: (i, 0)),
        pl.BlockSpec((256, 64), lambda i: (0, 0)),
    ],
    out_specs=pl.BlockSpec((_BLK, 64), lambda i: (i, 0)),
    out_shape=jax.ShapeDtypeStruct((N, 64), _F32),
)


def _tcs_body(xw_ref, d_ref, y_ref, dis_ref):
  deg = d_ref[0, :, 0:1] + d_ref[1, :, 0:1] + 1.0
  dis = lax.rsqrt(deg)
  y_ref[...] = xw_ref[...] * dis
  dis_ref[...] = dis


_tc_scale = pl.pallas_call(
    _tcs_body,
    grid=(_NBLK,),
    in_specs=[
        pl.BlockSpec((_BLK, 64), lambda i: (i, 0)),
        pl.BlockSpec((2, _BLK, 8), lambda i: (0, i, 0)),
    ],
    out_specs=[
        pl.BlockSpec((_BLK, 64), lambda i: (i, 0)),
        pl.BlockSpec((_BLK, 1), lambda i: (i, 0)),
    ],
    out_shape=[
        jax.ShapeDtypeStruct((N, 64), _F32),
        jax.ShapeDtypeStruct((N, 1), _F32),
    ],
)


def _tcb_body(p_ref, y1_ref, dis_ref, b1_ref, w2_ref, y2_ref):
  dis = dis_ref[...]
  acc = p_ref[0] + p_ref[1] + y1_ref[...]
  z = acc * dis + b1_ref[...]
  h = jnp.where(z >= 0, z, 0.01 * z)
  y2_ref[...] = jnp.dot(h, w2_ref[...], preferred_element_type=_F32) * dis


_tc_b = pl.pallas_call(
    _tcb_body,
    grid=(_NBLK,),
    in_specs=[
        pl.BlockSpec((2, _BLK, 64), lambda i: (0, i, 0)),
        pl.BlockSpec((_BLK, 64), lambda i: (i, 0)),
        pl.BlockSpec((_BLK, 1), lambda i: (i, 0)),
        pl.BlockSpec((1, 64), lambda i: (0, 0)),
        pl.BlockSpec((64, 16), lambda i: (0, 0)),
    ],
    out_specs=pl.BlockSpec((_BLK, 16), lambda i: (i, 0)),
    out_shape=jax.ShapeDtypeStruct((N, 16), _F32),
)


def _tcc_body(q_ref, y2_ref, dis_ref, b2_ref, o_ref):
  z = (q_ref[0] + q_ref[1] + y2_ref[...]) * dis_ref[...] + b2_ref[...]
  m = jnp.max(z, axis=1, keepdims=True)
  e = jnp.exp(z - m)
  o_ref[...] = (e / jnp.sum(e, axis=1, keepdims=True))[:, 0:4]


_tc_c = pl.pallas_call(
    _tcc_body,
    grid=(_NBLK,),
    in_specs=[
        pl.BlockSpec((2, _BLK, 16), lambda i: (0, i, 0)),
        pl.BlockSpec((_BLK, 16), lambda i: (i, 0)),
        pl.BlockSpec((_BLK, 1), lambda i: (i, 0)),
        pl.BlockSpec((1, 16), lambda i: (0, 0)),
    ],
    out_specs=pl.BlockSpec((_BLK, 4), lambda i: (i, 0)),
    out_shape=jax.ShapeDtypeStruct((N, 4), _F32),
)


def kernel(x, edge_index, W1, b1, W2, b2):
  srcs = edge_index[0].reshape(NW * NCHUNK, CHUNK)
  dsts = edge_index[1].reshape(NW * NCHUNK, CHUNK)
  zeros16 = jnp.zeros((NPAD, 16), _F32)
  zeros64 = jnp.zeros((NPAD, 64), _F32)
  zeros8 = jnp.zeros((NPAD, 8), _F32)
  ones8 = jnp.ones((CHUNK, 8), _F32)
  w2pad = jnp.pad(W2, ((0, 0), (0, 12)))
  b2pad = jnp.concatenate([b2, jnp.full((12,), -1e30, _F32)]).reshape(1, 16)

  d = _sc_degree(dsts, ones8, zeros8)
  xw = _tc_mm(x, W1)
  y1, dis = _tc_scale(xw, d)
  p = _sc_agg64(y1, srcs, dsts, zeros64)
  y2 = _tc_b(p, y1, dis, b1.reshape(1, 64), w2pad)
  q = _sc_agg16(y2, srcs, dsts, zeros16)
  return _tc_c(q, y2, dis, b2pad)


# speedup vs baseline: 1.3730x; 1.2152x over previous
import functools

import jax
import jax.numpy as jnp
from jax import lax
from jax.experimental import pallas as pl
from jax.experimental.pallas import tpu as pltpu
from jax.experimental.pallas import tpu_sc as plsc

N = 10000
E = 160000
NPAD = 10240
NC, NS = 2, 16
NW = NC * NS
EPT = E // NW
CHUNK = 125
NCHUNK = EPT // CHUNK
RPT = NPAD // NS

_F32 = jnp.float32


NBUF = 4
NGRP = NCHUNK // NBUF


_SPT = N // NS


def _make_sc_aggregate(D, stage, nbuf):
  mesh = plsc.VectorSubcoreMesh(core_axis_name="c", subcore_axis_name="s")
  ngrp = NCHUNK // nbuf
  tab_shape = (N, 128) if stage else (2 * N, 64)

  @functools.partial(
      pl.kernel,
      out_type=jax.ShapeDtypeStruct((NC, NPAD, 128), _F32),
      mesh=mesh,
      compiler_params=pltpu.CompilerParams(use_tc_tiling_on_sc=False),
      scratch_types=[
          pltpu.VMEM((NCHUNK, CHUNK), jnp.int32),
          pltpu.VMEM((NCHUNK, CHUNK), jnp.int32),
          [pltpu.VMEM((CHUNK, D), _F32) for _ in range(nbuf)],
          [pltpu.SemaphoreType.DMA for _ in range(nbuf)],
          [pltpu.SemaphoreType.DMA for _ in range(nbuf)],
      ] + ([pltpu.VMEM_SHARED((N, D), _F32)] if stage else []) + [
          pltpu.VMEM_SHARED((NPAD, D), _F32),
      ],
  )
  def agg(table_hbm, srcs_hbm, dsts_hbm, zeros_hbm, out_hbm,
          src_v, dst_v, bufs, gsems, ssems, *rest):
    tab_sh = rest[0] if stage else None
    acc_sh = rest[-1]
    c = lax.axis_index("c")
    s = lax.axis_index("s")
    wid = s * NC + c
    table = tab_sh if stage else table_hbm
    if stage:
      pltpu.sync_copy(table_hbm.at[pl.ds(s * _SPT, _SPT), pl.ds(0, D)],
                      tab_sh.at[pl.ds(s * _SPT, _SPT)])
    pltpu.sync_copy(zeros_hbm.at[pl.ds(s * RPT, RPT)],
                    acc_sh.at[pl.ds(s * RPT, RPT)])
    pltpu.sync_copy(srcs_hbm.at[pl.ds(wid * NCHUNK, NCHUNK)], src_v)
    pltpu.sync_copy(dsts_hbm.at[pl.ds(wid * NCHUNK, NCHUNK)], dst_v)
    plsc.subcore_barrier()

    for b in range(nbuf):
      pltpu.async_copy(table.at[src_v.at[b]], bufs[b], gsems[b])

    def body(g, carry):
      for b in range(nbuf):
        j = g * nbuf + b
        pltpu.make_async_copy(table.at[src_v.at[j]], bufs[b],
                              gsems[b]).wait()
        pltpu.async_copy(bufs[b], acc_sh.at[dst_v.at[j]], ssems[b], add=True)
      for b in range(nbuf):
        j = g * nbuf + b
        pltpu.make_async_copy(bufs[b], acc_sh.at[dst_v.at[j]],
                              ssems[b]).wait()

        @pl.when(g + 1 < ngrp)
        def _():
          pltpu.async_copy(table.at[src_v.at[j + nbuf]], bufs[b],
                           gsems[b])
      return carry

    lax.fori_loop(0, ngrp, body, 0)
    plsc.subcore_barrier()
    pltpu.sync_copy(acc_sh.at[pl.ds(s * RPT, RPT)],
                    out_hbm.at[c].at[pl.ds(s * RPT, RPT), pl.ds(0, D)])

  return agg


_sc_agg64 = _make_sc_aggregate(64, stage=False, nbuf=8)
_sc_agg16 = _make_sc_aggregate(16, stage=True, nbuf=4)

_deg_mesh = plsc.VectorSubcoreMesh(core_axis_name="c", subcore_axis_name="s")


@functools.partial(
    pl.kernel,
    out_type=jax.ShapeDtypeStruct((NC, NPAD, 128), _F32),
    mesh=_deg_mesh,
    compiler_params=pltpu.CompilerParams(use_tc_tiling_on_sc=False),
    scratch_types=[
        pltpu.VMEM((NCHUNK, CHUNK), jnp.int32),
        pltpu.VMEM((CHUNK, 16), _F32),
        pltpu.VMEM_SHARED((NPAD, 16), _F32),
        pltpu.SemaphoreType.DMA,
    ],
)
def _sc_degree(dsts_hbm, ones_hbm, zeros_hbm, out_hbm,
               dst_v, ones_v, deg_sh, sem):
  c = lax.axis_index("c")
  s = lax.axis_index("s")
  wid = s * NC + c
  pltpu.sync_copy(zeros_hbm.at[pl.ds(s * RPT, RPT)],
                  deg_sh.at[pl.ds(s * RPT, RPT)])
  pltpu.sync_copy(ones_hbm, ones_v)
  pltpu.sync_copy(dsts_hbm.at[pl.ds(wid * NCHUNK, NCHUNK)], dst_v)
  plsc.subcore_barrier()

  def body(j, carry):
    pltpu.async_copy(ones_v, deg_sh.at[dst_v.at[j]], sem, add=True)
    return carry

  lax.fori_loop(0, NCHUNK, body, 0)

  def drain(j, carry):
    pltpu.make_async_copy(ones_v, deg_sh.at[dst_v.at[j]], sem).wait()
    return carry

  lax.fori_loop(0, NCHUNK, drain, 0)
  plsc.subcore_barrier()
  pltpu.sync_copy(deg_sh.at[pl.ds(s * RPT, RPT)],
                  out_hbm.at[c].at[pl.ds(s * RPT, RPT), pl.ds(0, 16)])


_BLK = 2000
_NBLK = N // _BLK


def _tcmm_body(x_ref, w_ref, xw_ref):
  xw_ref[...] = jnp.dot(x_ref[...], w_ref[...], preferred_element_type=_F32)


_tc_mm = pl.pallas_call(
    _tcmm_body,
    grid=(_NBLK,),
    in_specs=[
        pl.BlockSpec((_BLK, 256), lambda i: (i, 0)),
        pl.BlockSpec((256, 64), lambda i: (0, 0)),
    ],
    out_specs=pl.BlockSpec((_BLK, 64), lambda i: (i, 0)),
    out_shape=jax.ShapeDtypeStruct((N, 64), _F32),
)


def _tcs_body(xw_ref, d_ref, y_ref, dis_ref):
  deg = d_ref[0, :, 0:1] + d_ref[1, :, 0:1] + 1.0
  dis = lax.rsqrt(deg)
  y = xw_ref[...] * dis
  y_ref[...] = jnp.concatenate([y, jnp.zeros_like(y)], axis=1)
  dis_ref[...] = dis


_tc_scale = pl.pallas_call(
    _tcs_body,
    grid=(_NBLK,),
    in_specs=[
        pl.BlockSpec((_BLK, 64), lambda i: (i, 0)),
        pl.BlockSpec((2, _BLK, 128), lambda i: (0, i, 0)),
    ],
    out_specs=[
        pl.BlockSpec((_BLK, 128), lambda i: (i, 0)),
        pl.BlockSpec((_BLK, 1), lambda i: (i, 0)),
    ],
    out_shape=[
        jax.ShapeDtypeStruct((N, 128), _F32),
        jax.ShapeDtypeStruct((N, 1), _F32),
    ],
)


def _tcb_body(p_ref, y1_ref, dis_ref, b1_ref, w2_ref, y2_ref):
  dis = dis_ref[...]
  acc = p_ref[0, :, 0:64] + p_ref[1, :, 0:64] + y1_ref[:, 0:64]
  z = acc * dis + b1_ref[...]
  h = jnp.where(z >= 0, z, 0.01 * z)
  y2_ref[...] = jnp.dot(h, w2_ref[...], preferred_element_type=_F32) * dis


_tc_b = pl.pallas_call(
    _tcb_body,
    grid=(_NBLK,),
    in_specs=[
        pl.BlockSpec((2, _BLK, 128), lambda i: (0, i, 0)),
        pl.BlockSpec((_BLK, 128), lambda i: (i, 0)),
        pl.BlockSpec((_BLK, 1), lambda i: (i, 0)),
        pl.BlockSpec((1, 64), lambda i: (0, 0)),
        pl.BlockSpec((64, 128), lambda i: (0, 0)),
    ],
    out_specs=pl.BlockSpec((_BLK, 128), lambda i: (i, 0)),
    out_shape=jax.ShapeDtypeStruct((N, 128), _F32),
)


def _tcc_body(q_ref, y2_ref, dis_ref, b2_ref, o_ref):
  z = (q_ref[0, :, 0:16] + q_ref[1, :, 0:16] + y2_ref[:, 0:16])
  z = z * dis_ref[...] + b2_ref[...]
  m = jnp.max(z, axis=1, keepdims=True)
  e = jnp.exp(z - m)
  o_ref[...] = (e / jnp.sum(e, axis=1, keepdims=True))[:, 0:4]


_tc_c = pl.pallas_call(
    _tcc_body,
    grid=(_NBLK,),
    in_specs=[
        pl.BlockSpec((2, _BLK, 128), lambda i: (0, i, 0)),
        pl.BlockSpec((_BLK, 128), lambda i: (i, 0)),
        pl.BlockSpec((_BLK, 1), lambda i: (i, 0)),
        pl.BlockSpec((1, 16), lambda i: (0, 0)),
    ],
    out_specs=pl.BlockSpec((_BLK, 4), lambda i: (i, 0)),
    out_shape=jax.ShapeDtypeStruct((N, 4), _F32),
)


def kernel(x, edge_index, W1, b1, W2, b2):
  srcs = edge_index[0].reshape(NW * NCHUNK, CHUNK)
  srcs2 = srcs * 2
  dsts = edge_index[1].reshape(NW * NCHUNK, CHUNK)
  zeros16 = jnp.zeros((NPAD, 16), _F32)
  zeros64 = jnp.zeros((NPAD, 64), _F32)
  ones16 = jnp.ones((CHUNK, 16), _F32)
  w2pad = jnp.pad(W2, ((0, 0), (0, 124)))
  b2pad = jnp.concatenate([b2, jnp.full((12,), -1e30, _F32)]).reshape(1, 16)

  d = _sc_degree(dsts, ones16, zeros16)
  xw = _tc_mm(x, W1)
  y1, dis = _tc_scale(xw, d)
  y1v = y1.reshape(2 * N, 64)
  p = _sc_agg64(y1v, srcs2, dsts, zeros64)
  y2 = _tc_b(p, y1, dis, b1.reshape(1, 64), w2pad)
  q = _sc_agg16(y2, srcs, dsts, zeros16)
  return _tc_c(q, y2, dis, b2pad)


# speedup vs baseline: 1.4194x; 1.0338x over previous
import functools

import jax
import jax.numpy as jnp
from jax import lax
from jax.experimental import pallas as pl
from jax.experimental.pallas import tpu as pltpu
from jax.experimental.pallas import tpu_sc as plsc

N = 10000
E = 160000
NPAD = 10240
NC, NS = 2, 16
NW = NC * NS
EPT = E // NW
CHUNK = 125
NCHUNK = EPT // CHUNK
RPT = NPAD // NS

_F32 = jnp.float32


NBUF = 4
NGRP = NCHUNK // NBUF


_SPT = N // NS


def _make_sc_aggregate(D, nbuf):
  mesh = plsc.VectorSubcoreMesh(core_axis_name="c", subcore_axis_name="s")
  ngrp = NCHUNK // nbuf

  @functools.partial(
      pl.kernel,
      out_type=jax.ShapeDtypeStruct((NC, NPAD, 128), _F32),
      mesh=mesh,
      compiler_params=pltpu.CompilerParams(use_tc_tiling_on_sc=False),
      scratch_types=[
          pltpu.VMEM((NCHUNK, CHUNK), jnp.int32),
          pltpu.VMEM((NCHUNK, CHUNK), jnp.int32),
          [pltpu.VMEM((CHUNK, D), _F32) for _ in range(nbuf)],
          [pltpu.SemaphoreType.DMA for _ in range(nbuf)],
          [pltpu.SemaphoreType.DMA for _ in range(nbuf)],
          pltpu.VMEM_SHARED((NPAD, D), _F32),
      ],
  )
  def agg(table_hbm, srcs_hbm, dsts_hbm, zeros_hbm, out_hbm,
          src_v, dst_v, bufs, gsems, ssems, acc_sh):
    c = lax.axis_index("c")
    s = lax.axis_index("s")
    wid = s * NC + c
    pltpu.sync_copy(zeros_hbm.at[pl.ds(s * RPT, RPT)],
                    acc_sh.at[pl.ds(s * RPT, RPT)])
    pltpu.sync_copy(srcs_hbm.at[pl.ds(wid * NCHUNK, NCHUNK)], src_v)
    pltpu.sync_copy(dsts_hbm.at[pl.ds(wid * NCHUNK, NCHUNK)], dst_v)
    plsc.subcore_barrier()

    for b in range(nbuf):
      pltpu.async_copy(table_hbm.at[src_v.at[b]], bufs[b], gsems[b])

    def body(g, carry):
      for b in range(nbuf):
        j = g * nbuf + b
        pltpu.make_async_copy(table_hbm.at[src_v.at[j]], bufs[b],
                              gsems[b]).wait()
        pltpu.async_copy(bufs[b], acc_sh.at[dst_v.at[j]], ssems[b], add=True)
      for b in range(nbuf):
        j = g * nbuf + b
        pltpu.make_async_copy(bufs[b], acc_sh.at[dst_v.at[j]],
                              ssems[b]).wait()

        @pl.when(g + 1 < ngrp)
        def _():
          pltpu.async_copy(table_hbm.at[src_v.at[j + nbuf]], bufs[b],
                           gsems[b])
      return carry

    lax.fori_loop(0, ngrp, body, 0)
    plsc.subcore_barrier()
    pltpu.sync_copy(acc_sh.at[pl.ds(s * RPT, RPT)],
                    out_hbm.at[c].at[pl.ds(s * RPT, RPT), pl.ds(0, D)])

  return agg


_sc_agg64 = _make_sc_aggregate(64, nbuf=8)
_sc_agg16 = _make_sc_aggregate(16, nbuf=8)

_deg_mesh = plsc.VectorSubcoreMesh(core_axis_name="c", subcore_axis_name="s")


@functools.partial(
    pl.kernel,
    out_type=jax.ShapeDtypeStruct((NC, NPAD, 128), _F32),
    mesh=_deg_mesh,
    compiler_params=pltpu.CompilerParams(use_tc_tiling_on_sc=False),
    scratch_types=[
        pltpu.VMEM((NCHUNK, CHUNK), jnp.int32),
        pltpu.VMEM((CHUNK, 16), _F32),
        pltpu.VMEM_SHARED((NPAD, 16), _F32),
        pltpu.SemaphoreType.DMA,
    ],
)
def _sc_degree(dsts_hbm, ones_hbm, zeros_hbm, out_hbm,
               dst_v, ones_v, deg_sh, sem):
  c = lax.axis_index("c")
  s = lax.axis_index("s")
  wid = s * NC + c
  pltpu.sync_copy(zeros_hbm.at[pl.ds(s * RPT, RPT)],
                  deg_sh.at[pl.ds(s * RPT, RPT)])
  pltpu.sync_copy(ones_hbm, ones_v)
  pltpu.sync_copy(dsts_hbm.at[pl.ds(wid * NCHUNK, NCHUNK)], dst_v)
  plsc.subcore_barrier()

  def body(j, carry):
    pltpu.async_copy(ones_v, deg_sh.at[dst_v.at[j]], sem, add=True)
    return carry

  lax.fori_loop(0, NCHUNK, body, 0)

  def drain(j, carry):
    pltpu.make_async_copy(ones_v, deg_sh.at[dst_v.at[j]], sem).wait()
    return carry

  lax.fori_loop(0, NCHUNK, drain, 0)
  plsc.subcore_barrier()
  pltpu.sync_copy(deg_sh.at[pl.ds(s * RPT, RPT)],
                  out_hbm.at[c].at[pl.ds(s * RPT, RPT), pl.ds(0, 16)])


_BLK = 2000
_NBLK = N // _BLK


def _tcmm_body(x_ref, w_ref, xw_ref):
  xw_ref[...] = jnp.dot(x_ref[...], w_ref[...], preferred_element_type=_F32)


_tc_mm = pl.pallas_call(
    _tcmm_body,
    grid=(_NBLK,),
    in_specs=[
        pl.BlockSpec((_BLK, 256), lambda i: (i, 0)),
        pl.BlockSpec((256, 64), lambda i: (0, 0)),
    ],
    out_specs=pl.BlockSpec((_BLK, 64), lambda i: (i, 0)),
    out_shape=jax.ShapeDtypeStruct((N, 64), _F32),
)


def _tcs_body(xw_ref, d_ref, y_ref, dis_ref):
  deg = d_ref[0, :, 0:1] + d_ref[1, :, 0:1] + 1.0
  dis = lax.rsqrt(deg)
  y = xw_ref[...] * dis
  y_ref[...] = jnp.concatenate([y, jnp.zeros_like(y)], axis=1)
  dis_ref[...] = dis


_tc_scale = pl.pallas_call(
    _tcs_body,
    grid=(_NBLK,),
    in_specs=[
        pl.BlockSpec((_BLK, 64), lambda i: (i, 0)),
        pl.BlockSpec((2, _BLK, 128), lambda i: (0, i, 0)),
    ],
    out_specs=[
        pl.BlockSpec((_BLK, 128), lambda i: (i, 0)),
        pl.BlockSpec((_BLK, 1), lambda i: (i, 0)),
    ],
    out_shape=[
        jax.ShapeDtypeStruct((N, 128), _F32),
        jax.ShapeDtypeStruct((N, 1), _F32),
    ],
)


def _tcb_body(p_ref, y1_ref, dis_ref, b1_ref, w2_ref, y2_ref):
  dis = dis_ref[...]
  acc = p_ref[0, :, 0:64] + p_ref[1, :, 0:64] + y1_ref[:, 0:64]
  z = acc * dis + b1_ref[...]
  h = jnp.where(z >= 0, z, 0.01 * z)
  y2_ref[...] = jnp.dot(h, w2_ref[...], preferred_element_type=_F32) * dis


_tc_b = pl.pallas_call(
    _tcb_body,
    grid=(_NBLK,),
    in_specs=[
        pl.BlockSpec((2, _BLK, 128), lambda i: (0, i, 0)),
        pl.BlockSpec((_BLK, 128), lambda i: (i, 0)),
        pl.BlockSpec((_BLK, 1), lambda i: (i, 0)),
        pl.BlockSpec((1, 64), lambda i: (0, 0)),
        pl.BlockSpec((64, 128), lambda i: (0, 0)),
    ],
    out_specs=pl.BlockSpec((_BLK, 128), lambda i: (i, 0)),
    out_shape=jax.ShapeDtypeStruct((N, 128), _F32),
)


def _tcc_body(q_ref, y2_ref, dis_ref, b2_ref, o_ref):
  z = (q_ref[0, :, 0:16] + q_ref[1, :, 0:16] + y2_ref[:, 0:16])
  z = z * dis_ref[...] + b2_ref[...]
  m = jnp.max(z, axis=1, keepdims=True)
  e = jnp.exp(z - m)
  o_ref[...] = (e / jnp.sum(e, axis=1, keepdims=True))[:, 0:4]


_tc_c = pl.pallas_call(
    _tcc_body,
    grid=(_NBLK,),
    in_specs=[
        pl.BlockSpec((2, _BLK, 128), lambda i: (0, i, 0)),
        pl.BlockSpec((_BLK, 128), lambda i: (i, 0)),
        pl.BlockSpec((_BLK, 1), lambda i: (i, 0)),
        pl.BlockSpec((1, 16), lambda i: (0, 0)),
    ],
    out_specs=pl.BlockSpec((_BLK, 4), lambda i: (i, 0)),
    out_shape=jax.ShapeDtypeStruct((N, 4), _F32),
)


def kernel(x, edge_index, W1, b1, W2, b2):
  srcs = edge_index[0].reshape(NW * NCHUNK, CHUNK)
  srcs2 = srcs * 2
  srcs8 = srcs * 8
  dsts = edge_index[1].reshape(NW * NCHUNK, CHUNK)
  zeros16 = jnp.zeros((NPAD, 16), _F32)
  zeros64 = jnp.zeros((NPAD, 64), _F32)
  ones16 = jnp.ones((CHUNK, 16), _F32)
  w2pad = jnp.pad(W2, ((0, 0), (0, 124)))
  b2pad = jnp.concatenate([b2, jnp.full((12,), -1e30, _F32)]).reshape(1, 16)

  d = _sc_degree(dsts, ones16, zeros16)
  xw = _tc_mm(x, W1)
  y1, dis = _tc_scale(xw, d)
  y1v = y1.reshape(2 * N, 64)
  p = _sc_agg64(y1v, srcs2, dsts, zeros64)
  y2 = _tc_b(p, y1, dis, b1.reshape(1, 64), w2pad)
  y2v = y2.reshape(8 * N, 16)
  q = _sc_agg16(y2v, srcs8, dsts, zeros16)
  return _tc_c(q, y2, dis, b2pad)


# speedup vs baseline: 1.4210x; 1.0011x over previous
import functools

import jax
import jax.numpy as jnp
from jax import lax
from jax.experimental import pallas as pl
from jax.experimental.pallas import tpu as pltpu
from jax.experimental.pallas import tpu_sc as plsc

N = 10000
E = 160000
NPAD = 10240
NC, NS = 2, 16
NW = NC * NS
EPT = E // NW
CHUNK = 125
NCHUNK = EPT // CHUNK
RPT = NPAD // NS

_F32 = jnp.float32


NBUF = 4
NGRP = NCHUNK // NBUF


_SPT = N // NS


def _make_sc_aggregate(D, nbuf):
  mesh = plsc.VectorSubcoreMesh(core_axis_name="c", subcore_axis_name="s")
  ngrp = NCHUNK // nbuf

  @functools.partial(
      pl.kernel,
      out_type=jax.ShapeDtypeStruct((NC, NPAD, 128), _F32),
      mesh=mesh,
      compiler_params=pltpu.CompilerParams(use_tc_tiling_on_sc=False),
      scratch_types=[
          pltpu.VMEM((NCHUNK, CHUNK), jnp.int32),
          pltpu.VMEM((NCHUNK, CHUNK), jnp.int32),
          [pltpu.VMEM((CHUNK, D), _F32) for _ in range(nbuf)],
          [pltpu.SemaphoreType.DMA for _ in range(nbuf)],
          [pltpu.SemaphoreType.DMA for _ in range(nbuf)],
          pltpu.VMEM_SHARED((NPAD, D), _F32),
      ],
  )
  def agg(table_hbm, srcs_hbm, dsts_hbm, zeros_hbm, out_hbm,
          src_v, dst_v, bufs, gsems, ssems, acc_sh):
    c = lax.axis_index("c")
    s = lax.axis_index("s")
    wid = s * NC + c
    pltpu.sync_copy(zeros_hbm.at[pl.ds(s * RPT, RPT)],
                    acc_sh.at[pl.ds(s * RPT, RPT)])
    pltpu.sync_copy(srcs_hbm.at[pl.ds(wid * NCHUNK, NCHUNK)], src_v)
    pltpu.sync_copy(dsts_hbm.at[pl.ds(wid * NCHUNK, NCHUNK)], dst_v)
    plsc.subcore_barrier()

    for b in range(nbuf):
      pltpu.async_copy(table_hbm.at[src_v.at[b]], bufs[b], gsems[b])

    def body(g, carry):
      for b in range(nbuf):
        j = g * nbuf + b
        pltpu.make_async_copy(table_hbm.at[src_v.at[j]], bufs[b],
                              gsems[b]).wait()
        pltpu.async_copy(bufs[b], acc_sh.at[dst_v.at[j]], ssems[b], add=True)
      for b in range(nbuf):
        j = g * nbuf + b
        pltpu.make_async_copy(bufs[b], acc_sh.at[dst_v.at[j]],
                              ssems[b]).wait()

        @pl.when(g + 1 < ngrp)
        def _():
          pltpu.async_copy(table_hbm.at[src_v.at[j + nbuf]], bufs[b],
                           gsems[b])
      return carry

    lax.fori_loop(0, ngrp, body, 0)
    plsc.subcore_barrier()
    pltpu.sync_copy(acc_sh.at[pl.ds(s * RPT, RPT)],
                    out_hbm.at[c].at[pl.ds(s * RPT, RPT), pl.ds(0, D)])

  return agg


_sc_agg64 = _make_sc_aggregate(64, nbuf=8)
_sc_agg16 = _make_sc_aggregate(16, nbuf=8)

_deg_mesh = plsc.VectorSubcoreMesh(core_axis_name="c", subcore_axis_name="s")


@functools.partial(
    pl.kernel,
    out_type=jax.ShapeDtypeStruct((NC, NPAD, 128), _F32),
    mesh=_deg_mesh,
    compiler_params=pltpu.CompilerParams(use_tc_tiling_on_sc=False),
    scratch_types=[
        pltpu.VMEM((NCHUNK, CHUNK), jnp.int32),
        pltpu.VMEM((CHUNK, 16), _F32),
        pltpu.VMEM_SHARED((NPAD, 16), _F32),
        pltpu.SemaphoreType.DMA,
    ],
)
def _sc_degree(dsts_hbm, ones_hbm, zeros_hbm, out_hbm,
               dst_v, ones_v, deg_sh, sem):
  c = lax.axis_index("c")
  s = lax.axis_index("s")
  wid = s * NC + c
  pltpu.sync_copy(zeros_hbm.at[pl.ds(s * RPT, RPT)],
                  deg_sh.at[pl.ds(s * RPT, RPT)])
  pltpu.sync_copy(ones_hbm, ones_v)
  pltpu.sync_copy(dsts_hbm.at[pl.ds(wid * NCHUNK, NCHUNK)], dst_v)
  plsc.subcore_barrier()

  def body(j, carry):
    pltpu.async_copy(ones_v, deg_sh.at[dst_v.at[j]], sem, add=True)
    return carry

  lax.fori_loop(0, NCHUNK, body, 0)

  def drain(j, carry):
    pltpu.make_async_copy(ones_v, deg_sh.at[dst_v.at[j]], sem).wait()
    return carry

  lax.fori_loop(0, NCHUNK, drain, 0)
  plsc.subcore_barrier()
  pltpu.sync_copy(deg_sh.at[pl.ds(s * RPT, RPT)],
                  out_hbm.at[c].at[pl.ds(s * RPT, RPT), pl.ds(0, 16)])


_BLK = 2000
_NBLK = N // _BLK


def _tcms_body(x_ref, w_ref, d_ref, y_ref, dis_ref):
  deg = d_ref[0, :, 0:1] + d_ref[1, :, 0:1] + 1.0
  dis = lax.rsqrt(deg)
  xw = jnp.dot(x_ref[...], w_ref[...], preferred_element_type=_F32)
  y = xw * dis
  y_ref[...] = jnp.concatenate([y, jnp.zeros_like(y)], axis=1)
  dis_ref[...] = dis


_tc_ms = pl.pallas_call(
    _tcms_body,
    grid=(_NBLK,),
    in_specs=[
        pl.BlockSpec((_BLK, 256), lambda i: (i, 0)),
        pl.BlockSpec((256, 64), lambda i: (0, 0)),
        pl.BlockSpec((2, _BLK, 128), lambda i: (0, i, 0)),
    ],
    out_specs=[
        pl.BlockSpec((_BLK, 128), lambda i: (i, 0)),
        pl.BlockSpec((_BLK, 1), lambda i: (i, 0)),
    ],
    out_shape=[
        jax.ShapeDtypeStruct((N, 128), _F32),
        jax.ShapeDtypeStruct((N, 1), _F32),
    ],
)


def _tcb_body(p_ref, y1_ref, dis_ref, b1_ref, w2_ref, y2_ref):
  dis = dis_ref[...]
  acc = p_ref[0, :, 0:64] + p_ref[1, :, 0:64] + y1_ref[:, 0:64]
  z = acc * dis + b1_ref[...]
  h = jnp.where(z >= 0, z, 0.01 * z)
  y2_ref[...] = jnp.dot(h, w2_ref[...], preferred_element_type=_F32) * dis


_tc_b = pl.pallas_call(
    _tcb_body,
    grid=(_NBLK,),
    in_specs=[
        pl.BlockSpec((2, _BLK, 128), lambda i: (0, i, 0)),
        pl.BlockSpec((_BLK, 128), lambda i: (i, 0)),
        pl.BlockSpec((_BLK, 1), lambda i: (i, 0)),
        pl.BlockSpec((1, 64), lambda i: (0, 0)),
        pl.BlockSpec((64, 128), lambda i: (0, 0)),
    ],
    out_specs=pl.BlockSpec((_BLK, 128), lambda i: (i, 0)),
    out_shape=jax.ShapeDtypeStruct((N, 128), _F32),
)


def _tcc_body(q_ref, y2_ref, dis_ref, b2_ref, o_ref):
  z = (q_ref[0, :, 0:16] + q_ref[1, :, 0:16] + y2_ref[:, 0:16])
  z = z * dis_ref[...] + b2_ref[...]
  m = jnp.max(z, axis=1, keepdims=True)
  e = jnp.exp(z - m)
  o_ref[...] = (e / jnp.sum(e, axis=1, keepdims=True))[:, 0:4]


_tc_c = pl.pallas_call(
    _tcc_body,
    grid=(_NBLK,),
    in_specs=[
        pl.BlockSpec((2, _BLK, 128), lambda i: (0, i, 0)),
        pl.BlockSpec((_BLK, 128), lambda i: (i, 0)),
        pl.BlockSpec((_BLK, 1), lambda i: (i, 0)),
        pl.BlockSpec((1, 16), lambda i: (0, 0)),
    ],
    out_specs=pl.BlockSpec((_BLK, 4), lambda i: (i, 0)),
    out_shape=jax.ShapeDtypeStruct((N, 4), _F32),
)


def kernel(x, edge_index, W1, b1, W2, b2):
  srcs = edge_index[0].reshape(NW * NCHUNK, CHUNK)
  srcs2 = srcs * 2
  srcs8 = srcs * 8
  dsts = edge_index[1].reshape(NW * NCHUNK, CHUNK)
  zeros16 = jnp.zeros((NPAD, 16), _F32)
  zeros64 = jnp.zeros((NPAD, 64), _F32)
  ones16 = jnp.ones((CHUNK, 16), _F32)
  w2pad = jnp.pad(W2, ((0, 0), (0, 124)))
  b2pad = jnp.concatenate([b2, jnp.full((12,), -1e30, _F32)]).reshape(1, 16)

  d = _sc_degree(dsts, ones16, zeros16)
  y1, dis = _tc_ms(x, W1, d)
  y1v = y1.reshape(2 * N, 64)
  p = _sc_agg64(y1v, srcs2, dsts, zeros64)
  y2 = _tc_b(p, y1, dis, b1.reshape(1, 64), w2pad)
  y2v = y2.reshape(8 * N, 16)
  q = _sc_agg16(y2v, srcs8, dsts, zeros16)
  return _tc_c(q, y2, dis, b2pad)


# speedup vs baseline: 1.4230x; 1.0014x over previous
import functools

import jax
import jax.numpy as jnp
from jax import lax
from jax.experimental import pallas as pl
from jax.experimental.pallas import tpu as pltpu
from jax.experimental.pallas import tpu_sc as plsc

N = 10000
E = 160000
NPAD = 10240
NC, NS = 2, 16
NW = NC * NS
EPT = E // NW
CHUNK = 125
NCHUNK = EPT // CHUNK
RPT = NPAD // NS

_F32 = jnp.float32


NBUF = 4
NGRP = NCHUNK // NBUF


_SPT = N // NS


def _make_sc_aggregate(D, nbuf):
  mesh = plsc.VectorSubcoreMesh(core_axis_name="c", subcore_axis_name="s")
  ngrp = NCHUNK // nbuf

  @functools.partial(
      pl.kernel,
      out_type=jax.ShapeDtypeStruct((NC, NPAD, 128), _F32),
      mesh=mesh,
      compiler_params=pltpu.CompilerParams(use_tc_tiling_on_sc=False),
      scratch_types=[
          pltpu.VMEM((NCHUNK, CHUNK), jnp.int32),
          pltpu.VMEM((NCHUNK, CHUNK), jnp.int32),
          [pltpu.VMEM((CHUNK, D), _F32) for _ in range(nbuf)],
          [pltpu.SemaphoreType.DMA for _ in range(nbuf)],
          [pltpu.SemaphoreType.DMA for _ in range(nbuf)],
          pltpu.VMEM_SHARED((NPAD, D), _F32),
      ],
  )
  def agg(table_hbm, srcs_hbm, dsts_hbm, zeros_hbm, out_hbm,
          src_v, dst_v, bufs, gsems, ssems, acc_sh):
    c = lax.axis_index("c")
    s = lax.axis_index("s")
    wid = s * NC + c
    pltpu.sync_copy(zeros_hbm.at[pl.ds(s * RPT, RPT)],
                    acc_sh.at[pl.ds(s * RPT, RPT)])
    pltpu.sync_copy(srcs_hbm.at[pl.ds(wid * NCHUNK, NCHUNK)], src_v)
    pltpu.sync_copy(dsts_hbm.at[pl.ds(wid * NCHUNK, NCHUNK)], dst_v)
    plsc.subcore_barrier()

    for b in range(nbuf):
      pltpu.async_copy(table_hbm.at[src_v.at[b]], bufs[b], gsems[b])

    def body(g, carry):
      for b in range(nbuf):
        j = g * nbuf + b
        pltpu.make_async_copy(table_hbm.at[src_v.at[j]], bufs[b],
                              gsems[b]).wait()
        pltpu.async_copy(bufs[b], acc_sh.at[dst_v.at[j]], ssems[b], add=True)
      for b in range(nbuf):
        j = g * nbuf + b
        pltpu.make_async_copy(bufs[b], acc_sh.at[dst_v.at[j]],
                              ssems[b]).wait()

        @pl.when(g + 1 < ngrp)
        def _():
          pltpu.async_copy(table_hbm.at[src_v.at[j + nbuf]], bufs[b],
                           gsems[b])
      return carry

    lax.fori_loop(0, ngrp, body, 0)
    plsc.subcore_barrier()
    pltpu.sync_copy(acc_sh.at[pl.ds(s * RPT, RPT)],
                    out_hbm.at[c].at[pl.ds(s * RPT, RPT), pl.ds(0, D)])

  return agg


_sc_agg64 = _make_sc_aggregate(64, nbuf=8)
_sc_agg16 = _make_sc_aggregate(16, nbuf=10)

_deg_mesh = plsc.VectorSubcoreMesh(core_axis_name="c", subcore_axis_name="s")


@functools.partial(
    pl.kernel,
    out_type=jax.ShapeDtypeStruct((NC, NPAD, 128), _F32),
    mesh=_deg_mesh,
    compiler_params=pltpu.CompilerParams(use_tc_tiling_on_sc=False),
    scratch_types=[
        pltpu.VMEM((NCHUNK, CHUNK), jnp.int32),
        pltpu.VMEM((CHUNK, 16), _F32),
        pltpu.VMEM_SHARED((NPAD, 16), _F32),
        pltpu.SemaphoreType.DMA,
    ],
)
def _sc_degree(dsts_hbm, ones_hbm, zeros_hbm, out_hbm,
               dst_v, ones_v, deg_sh, sem):
  c = lax.axis_index("c")
  s = lax.axis_index("s")
  wid = s * NC + c
  pltpu.sync_copy(zeros_hbm.at[pl.ds(s * RPT, RPT)],
                  deg_sh.at[pl.ds(s * RPT, RPT)])
  pltpu.sync_copy(ones_hbm, ones_v)
  pltpu.sync_copy(dsts_hbm.at[pl.ds(wid * NCHUNK, NCHUNK)], dst_v)
  plsc.subcore_barrier()

  def body(j, carry):
    pltpu.async_copy(ones_v, deg_sh.at[dst_v.at[j]], sem, add=True)
    return carry

  lax.fori_loop(0, NCHUNK, body, 0)

  def drain(j, carry):
    pltpu.make_async_copy(ones_v, deg_sh.at[dst_v.at[j]], sem).wait()
    return carry

  lax.fori_loop(0, NCHUNK, drain, 0)
  plsc.subcore_barrier()
  pltpu.sync_copy(deg_sh.at[pl.ds(s * RPT, RPT)],
                  out_hbm.at[c].at[pl.ds(s * RPT, RPT), pl.ds(0, 16)])


_BLK = 2000
_NBLK = N // _BLK


def _tcms_body(x_ref, w_ref, d_ref, y_ref, dis_ref):
  deg = d_ref[0, :, 0:1] + d_ref[1, :, 0:1] + 1.0
  dis = lax.rsqrt(deg)
  xw = jnp.dot(x_ref[...], w_ref[...], preferred_element_type=_F32)
  y = xw * dis
  y_ref[...] = jnp.concatenate([y, jnp.zeros_like(y)], axis=1)
  dis_ref[...] = dis


_tc_ms = pl.pallas_call(
    _tcms_body,
    grid=(_NBLK,),
    in_specs=[
        pl.BlockSpec((_BLK, 256), lambda i: (i, 0)),
        pl.BlockSpec((256, 64), lambda i: (0, 0)),
        pl.BlockSpec((2, _BLK, 128), lambda i: (0, i, 0)),
    ],
    out_specs=[
        pl.BlockSpec((_BLK, 128), lambda i: (i, 0)),
        pl.BlockSpec((_BLK, 1), lambda i: (i, 0)),
    ],
    out_shape=[
        jax.ShapeDtypeStruct((N, 128), _F32),
        jax.ShapeDtypeStruct((N, 1), _F32),
    ],
)


def _tcb_body(p_ref, y1_ref, dis_ref, b1_ref, w2_ref, y2_ref):
  dis = dis_ref[...]
  acc = p_ref[0, :, 0:64] + p_ref[1, :, 0:64] + y1_ref[:, 0:64]
  z = acc * dis + b1_ref[...]
  h = jnp.where(z >= 0, z, 0.01 * z)
  y2_ref[...] = jnp.dot(h, w2_ref[...], preferred_element_type=_F32) * dis


_tc_b = pl.pallas_call(
    _tcb_body,
    grid=(_NBLK,),
    in_specs=[
        pl.BlockSpec((2, _BLK, 128), lambda i: (0, i, 0)),
        pl.BlockSpec((_BLK, 128), lambda i: (i, 0)),
        pl.BlockSpec((_BLK, 1), lambda i: (i, 0)),
        pl.BlockSpec((1, 64), lambda i: (0, 0)),
        pl.BlockSpec((64, 128), lambda i: (0, 0)),
    ],
    out_specs=pl.BlockSpec((_BLK, 128), lambda i: (i, 0)),
    out_shape=jax.ShapeDtypeStruct((N, 128), _F32),
)


def _tcc_body(q_ref, y2_ref, dis_ref, b2_ref, o_ref):
  z = (q_ref[0, :, 0:16] + q_ref[1, :, 0:16] + y2_ref[:, 0:16])
  z = z * dis_ref[...] + b2_ref[...]
  m = jnp.max(z, axis=1, keepdims=True)
  e = jnp.exp(z - m)
  o_ref[...] = (e / jnp.sum(e, axis=1, keepdims=True))[:, 0:4]


_tc_c = pl.pallas_call(
    _tcc_body,
    grid=(_NBLK,),
    in_specs=[
        pl.BlockSpec((2, _BLK, 128), lambda i: (0, i, 0)),
        pl.BlockSpec((_BLK, 128), lambda i: (i, 0)),
        pl.BlockSpec((_BLK, 1), lambda i: (i, 0)),
        pl.BlockSpec((1, 16), lambda i: (0, 0)),
    ],
    out_specs=pl.BlockSpec((_BLK, 4), lambda i: (i, 0)),
    out_shape=jax.ShapeDtypeStruct((N, 4), _F32),
)


def kernel(x, edge_index, W1, b1, W2, b2):
  srcs = edge_index[0].reshape(NW * NCHUNK, CHUNK)
  srcs2 = srcs * 2
  srcs8 = srcs * 8
  dsts = edge_index[1].reshape(NW * NCHUNK, CHUNK)
  zeros16 = jnp.zeros((NPAD, 16), _F32)
  zeros64 = jnp.zeros((NPAD, 64), _F32)
  ones16 = jnp.ones((CHUNK, 16), _F32)
  w2pad = jnp.pad(W2, ((0, 0), (0, 124)))
  b2pad = jnp.concatenate([b2, jnp.full((12,), -1e30, _F32)]).reshape(1, 16)

  d = _sc_degree(dsts, ones16, zeros16)
  y1, dis = _tc_ms(x, W1, d)
  y1v = y1.reshape(2 * N, 64)
  p = _sc_agg64(y1v, srcs2, dsts, zeros64)
  y2 = _tc_b(p, y1, dis, b1.reshape(1, 64), w2pad)
  y2v = y2.reshape(8 * N, 16)
  q = _sc_agg16(y2v, srcs8, dsts, zeros16)
  return _tc_c(q, y2, dis, b2pad)
